# Initial kernel scaffold; baseline (speedup 1.0000x reference)
import functools

import jax
import jax.numpy as jnp
from jax import lax
from jax.experimental import pallas as pl
from jax.experimental.pallas import tpu as pltpu
from jax.experimental.pallas import tpu_sc as plsc

_ROW_BLOCK = 2000


def _linear_body(x_ref, w_ref, b_ref, o_ref):
    o_ref[...] = lax.dot_general(
        x_ref[...], w_ref[...], (((1,), (1,)), ((), ())),
        preferred_element_type=jnp.float32,
        precision=lax.Precision.HIGHEST,
    ) + b_ref[...]


def _linear(x, w, b):
    n, d = x.shape
    br = _ROW_BLOCK
    return pl.pallas_call(
        _linear_body,
        grid=(n // br,),
        in_specs=[
            pl.BlockSpec((br, d), lambda i: (i, 0)),
            pl.BlockSpec((d, d), lambda i: (0, 0)),
            pl.BlockSpec((1, d), lambda i: (0, 0)),
        ],
        out_specs=pl.BlockSpec((br, d), lambda i: (i, 0)),
        out_shape=jax.ShapeDtypeStruct((n, d), jnp.float32),
    )(x, w, b.reshape(1, d))


def _mid_body(p_ref, y_ref, w_ref, b_ref, o_ref):
    h = p_ref[0] + p_ref[1] + y_ref[...]
    h = jnp.maximum(h, 0.0)
    o_ref[...] = lax.dot_general(
        h, w_ref[...], (((1,), (1,)), ((), ())),
        preferred_element_type=jnp.float32,
        precision=lax.Precision.HIGHEST,
    ) + b_ref[...]


def _mid(p, y, w, b):
    n, d = y.shape
    br = _ROW_BLOCK
    return pl.pallas_call(
        _mid_body,
        grid=(n // br,),
        in_specs=[
            pl.BlockSpec((2, br, d), lambda i: (0, i, 0)),
            pl.BlockSpec((br, d), lambda i: (i, 0)),
            pl.BlockSpec((d, d), lambda i: (0, 0)),
            pl.BlockSpec((1, d), lambda i: (0, 0)),
        ],
        out_specs=pl.BlockSpec((br, d), lambda i: (i, 0)),
        out_shape=jax.ShapeDtypeStruct((n, d), jnp.float32),
    )(p, y, w, b.reshape(1, d))


def _final_body(p_ref, y_ref, o_ref):
    z = p_ref[0] + p_ref[1] + y_ref[...]
    m = jnp.max(z, axis=1, keepdims=True)
    s = z - m
    lse = jnp.log(jnp.sum(jnp.exp(s), axis=1, keepdims=True))
    o_ref[...] = s - lse


def _final(p, y):
    n, d = y.shape
    br = _ROW_BLOCK
    return pl.pallas_call(
        _final_body,
        grid=(n // br,),
        in_specs=[
            pl.BlockSpec((2, br, d), lambda i: (0, i, 0)),
            pl.BlockSpec((br, d), lambda i: (i, 0)),
        ],
        out_specs=pl.BlockSpec((br, d), lambda i: (i, 0)),
        out_shape=jax.ShapeDtypeStruct((n, d), jnp.float32),
    )(p, y)


@functools.lru_cache(maxsize=None)
def _make_scatter(n, d, e):
    info = plsc.get_sparse_core_info()
    nc, ns = info.num_cores, info.num_subcores
    k = 80
    epw = e // (nc * ns)
    chunks = epw // k
    rpt = n // ns
    mesh = plsc.VectorSubcoreMesh(core_axis_name="c", subcore_axis_name="s")

    @functools.partial(
        pl.kernel, mesh=mesh,
        out_type=jax.ShapeDtypeStruct((nc, n, d), jnp.float32),
        scratch_types=[
            pltpu.VMEM((k,), jnp.int32),
            pltpu.VMEM((k,), jnp.int32),
            pltpu.VMEM((k, d), jnp.float32),
            pltpu.VMEM_SHARED((n, d), jnp.float32),
            pltpu.SemaphoreType.DMA,
        ],
    )
    def scatter(y_hbm, row_hbm, col_hbm, zeros_hbm, out_hbm,
                ridx, cidx, rows_v, acc, sem):
        cid = lax.axis_index("c")
        sid = lax.axis_index("s")
        r0 = sid * rpt
        pltpu.sync_copy(zeros_hbm.at[pl.ds(r0, rpt)], acc.at[pl.ds(r0, rpt)])
        plsc.subcore_barrier()
        base0 = cid * (e // nc) + sid * epw

        def body(c, carry):
            b = base0 + c * k
            pltpu.sync_copy(row_hbm.at[pl.ds(b, k)], ridx)
            pltpu.sync_copy(col_hbm.at[pl.ds(b, k)], cidx)
            pltpu.async_copy(y_hbm.at[ridx], rows_v, sem).wait()
            pltpu.sync_copy(rows_v, acc.at[cidx], add=True)
            return carry

        lax.fori_loop(0, chunks, body, 0)
        plsc.subcore_barrier()
        pltpu.sync_copy(acc.at[pl.ds(r0, rpt)], out_hbm.at[cid, pl.ds(r0, rpt)])

    return scatter


def kernel(x, edge_index, W1, b1, W2, b2):
    n, d = x.shape
    e = edge_index.shape[1]
    row = edge_index[0]
    col = edge_index[1]
    zeros = jnp.zeros((n, d), jnp.float32)
    scatter = _make_scatter(n, d, e)

    y1 = _linear(x, W1, b1)
    p1 = scatter(y1, row, col, zeros)
    y2 = _mid(p1, y1, W2, b2)
    p2 = scatter(y2, row, col, zeros)
    return _final(p2, y2)


# speedup vs baseline: 7.3335x; 7.3335x over previous
import functools

import jax
import jax.numpy as jnp
from jax import lax
from jax.experimental import pallas as pl
from jax.experimental.pallas import tpu as pltpu
from jax.experimental.pallas import tpu_sc as plsc

_ROW_BLOCK = 2000


def _linear_body(x_ref, w_ref, b_ref, o_ref):
    o_ref[...] = lax.dot_general(
        x_ref[...], w_ref[...], (((1,), (1,)), ((), ())),
        preferred_element_type=jnp.float32,
        precision=lax.Precision.HIGHEST,
    ) + b_ref[...]


def _linear(x, w, b):
    n, d = x.shape
    br = _ROW_BLOCK
    return pl.pallas_call(
        _linear_body,
        grid=(n // br,),
        in_specs=[
            pl.BlockSpec((br, d), lambda i: (i, 0)),
            pl.BlockSpec((d, d), lambda i: (0, 0)),
            pl.BlockSpec((1, d), lambda i: (0, 0)),
        ],
        out_specs=pl.BlockSpec((br, d), lambda i: (i, 0)),
        out_shape=jax.ShapeDtypeStruct((n, d), jnp.float32),
    )(x, w, b.reshape(1, d))


def _mid_body(p_ref, y_ref, w_ref, b_ref, o_ref):
    h = p_ref[0] + p_ref[1] + y_ref[...]
    h = jnp.maximum(h, 0.0)
    o_ref[...] = lax.dot_general(
        h, w_ref[...], (((1,), (1,)), ((), ())),
        preferred_element_type=jnp.float32,
        precision=lax.Precision.HIGHEST,
    ) + b_ref[...]


def _mid(p, y, w, b):
    n, d = y.shape
    br = _ROW_BLOCK
    return pl.pallas_call(
        _mid_body,
        grid=(n // br,),
        in_specs=[
            pl.BlockSpec((2, br, d), lambda i: (0, i, 0)),
            pl.BlockSpec((br, d), lambda i: (i, 0)),
            pl.BlockSpec((d, d), lambda i: (0, 0)),
            pl.BlockSpec((1, d), lambda i: (0, 0)),
        ],
        out_specs=pl.BlockSpec((br, d), lambda i: (i, 0)),
        out_shape=jax.ShapeDtypeStruct((n, d), jnp.float32),
    )(p, y, w, b.reshape(1, d))


def _final_body(p_ref, y_ref, o_ref):
    z = p_ref[0] + p_ref[1] + y_ref[...]
    m = jnp.max(z, axis=1, keepdims=True)
    s = z - m
    lse = jnp.log(jnp.sum(jnp.exp(s), axis=1, keepdims=True))
    o_ref[...] = s - lse


def _final(p, y):
    n, d = y.shape
    br = _ROW_BLOCK
    return pl.pallas_call(
        _final_body,
        grid=(n // br,),
        in_specs=[
            pl.BlockSpec((2, br, d), lambda i: (0, i, 0)),
            pl.BlockSpec((br, d), lambda i: (i, 0)),
        ],
        out_specs=pl.BlockSpec((br, d), lambda i: (i, 0)),
        out_shape=jax.ShapeDtypeStruct((n, d), jnp.float32),
    )(p, y)


@functools.lru_cache(maxsize=None)
def _make_scatter(n, d, e):
    info = plsc.get_sparse_core_info()
    nc, ns = info.num_cores, info.num_subcores
    k = 80
    epw = e // (nc * ns)
    chunks = epw // k
    slab = (n // (8 * ns)) * 8
    tail = n - slab * ns
    mesh = plsc.VectorSubcoreMesh(core_axis_name="c", subcore_axis_name="s")

    @functools.partial(
        pl.kernel, mesh=mesh,
        out_type=jax.ShapeDtypeStruct((nc, n, d), jnp.float32),
        scratch_types=[
            pltpu.VMEM((k,), jnp.int32),
            pltpu.VMEM((k,), jnp.int32),
            pltpu.VMEM((k, d), jnp.float32),
            pltpu.VMEM_SHARED((n, d), jnp.float32),
            pltpu.SemaphoreType.DMA,
        ],
    )
    def scatter(y_hbm, row_hbm, col_hbm, zeros_hbm, out_hbm,
                ridx, cidx, rows_v, acc, sem):
        cid = lax.axis_index("c")
        sid = lax.axis_index("s")
        r0 = pl.multiple_of(sid * slab, 8)
        pltpu.sync_copy(zeros_hbm.at[pl.ds(r0, slab)], acc.at[pl.ds(r0, slab)])
        if tail:
            @pl.when(sid == 0)
            def _zero_tail():
                pltpu.sync_copy(zeros_hbm.at[pl.ds(slab * ns, tail)],
                                acc.at[pl.ds(slab * ns, tail)])
        plsc.subcore_barrier()
        base0 = cid * (e // nc) + sid * epw

        def body(c, carry):
            b = base0 + c * k
            pltpu.sync_copy(row_hbm.at[pl.ds(b, k)], ridx)
            pltpu.sync_copy(col_hbm.at[pl.ds(b, k)], cidx)
            pltpu.async_copy(y_hbm.at[ridx], rows_v, sem).wait()
            pltpu.sync_copy(rows_v, acc.at[cidx], add=True)
            return carry

        lax.fori_loop(0, chunks, body, 0)
        plsc.subcore_barrier()
        pltpu.sync_copy(acc.at[pl.ds(r0, slab)], out_hbm.at[cid, pl.ds(r0, slab)])
        if tail:
            @pl.when(sid == 0)
            def _write_tail():
                pltpu.sync_copy(acc.at[pl.ds(slab * ns, tail)],
                                out_hbm.at[cid, pl.ds(slab * ns, tail)])

    return scatter


def kernel(x, edge_index, W1, b1, W2, b2):
    n, d = x.shape
    e = edge_index.shape[1]
    row = edge_index[0]
    col = edge_index[1]
    zeros = jnp.zeros((n, d), jnp.float32)
    scatter = _make_scatter(n, d, e)

    y1 = _linear(x, W1, b1)
    p1 = scatter(y1, row, col, zeros)
    y2 = _mid(p1, y1, W2, b2)
    p2 = scatter(y2, row, col, zeros)
    return _final(p2, y2)


# speedup vs baseline: 13.1990x; 1.7998x over previous
import functools

import jax
import jax.numpy as jnp
from jax import lax
from jax.experimental import pallas as pl
from jax.experimental.pallas import tpu as pltpu
from jax.experimental.pallas import tpu_sc as plsc

_ROW_BLOCK = 2000


def _linear_body(x_ref, w_ref, b_ref, o_ref):
    o_ref[...] = lax.dot_general(
        x_ref[...], w_ref[...], (((1,), (1,)), ((), ())),
        preferred_element_type=jnp.float32,
        precision=lax.Precision.HIGHEST,
    ) + b_ref[...]


def _linear(x, w, b):
    n, d = x.shape
    br = _ROW_BLOCK
    return pl.pallas_call(
        _linear_body,
        grid=(n // br,),
        in_specs=[
            pl.BlockSpec((br, d), lambda i: (i, 0)),
            pl.BlockSpec((d, d), lambda i: (0, 0)),
            pl.BlockSpec((1, d), lambda i: (0, 0)),
        ],
        out_specs=pl.BlockSpec((br, d), lambda i: (i, 0)),
        out_shape=jax.ShapeDtypeStruct((n, d), jnp.float32),
    )(x, w, b.reshape(1, d))


def _mid_body(p_ref, y_ref, w_ref, b_ref, o_ref):
    h = p_ref[0] + p_ref[1] + y_ref[...]
    h = jnp.maximum(h, 0.0)
    o_ref[...] = lax.dot_general(
        h, w_ref[...], (((1,), (1,)), ((), ())),
        preferred_element_type=jnp.float32,
        precision=lax.Precision.HIGHEST,
    ) + b_ref[...]


def _mid(p, y, w, b):
    n, d = y.shape
    br = _ROW_BLOCK
    return pl.pallas_call(
        _mid_body,
        grid=(n // br,),
        in_specs=[
            pl.BlockSpec((2, br, d), lambda i: (0, i, 0)),
            pl.BlockSpec((br, d), lambda i: (i, 0)),
            pl.BlockSpec((d, d), lambda i: (0, 0)),
            pl.BlockSpec((1, d), lambda i: (0, 0)),
        ],
        out_specs=pl.BlockSpec((br, d), lambda i: (i, 0)),
        out_shape=jax.ShapeDtypeStruct((n, d), jnp.float32),
    )(p, y, w, b.reshape(1, d))


def _final_body(p_ref, y_ref, o_ref):
    z = p_ref[0] + p_ref[1] + y_ref[...]
    m = jnp.max(z, axis=1, keepdims=True)
    s = z - m
    lse = jnp.log(jnp.sum(jnp.exp(s), axis=1, keepdims=True))
    o_ref[...] = s - lse


def _final(p, y):
    n, d = y.shape
    br = _ROW_BLOCK
    return pl.pallas_call(
        _final_body,
        grid=(n // br,),
        in_specs=[
            pl.BlockSpec((2, br, d), lambda i: (0, i, 0)),
            pl.BlockSpec((br, d), lambda i: (i, 0)),
        ],
        out_specs=pl.BlockSpec((br, d), lambda i: (i, 0)),
        out_shape=jax.ShapeDtypeStruct((n, d), jnp.float32),
    )(p, y)


@functools.lru_cache(maxsize=None)
def _make_scatter(n, d, e):
    info = plsc.get_sparse_core_info()
    nc, ns = info.num_cores, info.num_subcores
    k = 80
    epw = e // (nc * ns)
    chunks = epw // k
    slab = (n // (8 * ns)) * 8
    tail = n - slab * ns
    mesh = plsc.VectorSubcoreMesh(core_axis_name="c", subcore_axis_name="s")

    assert chunks % 2 == 1

    @functools.partial(
        pl.kernel, mesh=mesh,
        out_type=jax.ShapeDtypeStruct((nc, n, d), jnp.float32),
        scratch_types=[
            pltpu.VMEM((epw,), jnp.int32),
            pltpu.VMEM((chunks, k), jnp.int32),
            pltpu.VMEM((k, d), jnp.float32),
            pltpu.VMEM((k, d), jnp.float32),
            pltpu.VMEM_SHARED((n, d), jnp.float32),
            pltpu.SemaphoreType.DMA,
            pltpu.SemaphoreType.DMA,
        ],
    )
    def scatter(y_hbm, row_hbm, col3_hbm, zeros_hbm, out_hbm,
                ridx_all, cidx_all, rows0, rows1, acc, sem0, sem1):
        cid = lax.axis_index("c")
        sid = lax.axis_index("s")
        wid = cid * ns + sid
        wbase = wid * epw
        pltpu.sync_copy(row_hbm.at[pl.ds(wbase, epw)], ridx_all)
        pltpu.sync_copy(col3_hbm.at[wid], cidx_all)
        r0 = pl.multiple_of(sid * slab, 8)
        pltpu.sync_copy(zeros_hbm.at[pl.ds(r0, slab)], acc.at[pl.ds(r0, slab)])
        if tail:
            @pl.when(sid == 0)
            def _zero_tail():
                pltpu.sync_copy(zeros_hbm.at[pl.ds(slab * ns, tail)],
                                acc.at[pl.ds(slab * ns, tail)])
        plsc.subcore_barrier()

        def gather(c, buf, sem):
            return pltpu.async_copy(
                y_hbm.at[ridx_all.at[pl.ds(c * k, k)]], buf, sem)

        def scat(c, buf):
            pltpu.sync_copy(buf, acc.at[cidx_all.at[c]], add=True)

        gather(0, rows0, sem0)

        def body(g, carry):
            c = 2 * g
            pltpu.make_async_copy(y_hbm.at[pl.ds(0, k)], rows0, sem0).wait()
            gather(c + 1, rows1, sem1)
            scat(c, rows0)
            pltpu.make_async_copy(y_hbm.at[pl.ds(0, k)], rows1, sem1).wait()
            gather(c + 2, rows0, sem0)
            scat(c + 1, rows1)
            return carry

        lax.fori_loop(0, chunks // 2, body, 0)
        pltpu.make_async_copy(y_hbm.at[pl.ds(0, k)], rows0, sem0).wait()
        scat(chunks - 1, rows0)

        plsc.subcore_barrier()
        pltpu.sync_copy(acc.at[pl.ds(r0, slab)], out_hbm.at[cid, pl.ds(r0, slab)])
        if tail:
            @pl.when(sid == 0)
            def _write_tail():
                pltpu.sync_copy(acc.at[pl.ds(slab * ns, tail)],
                                out_hbm.at[cid, pl.ds(slab * ns, tail)])

    def run(y, row, col, zeros):
        return scatter(y, row, col.reshape(nc * ns, chunks, k), zeros)

    return run


def kernel(x, edge_index, W1, b1, W2, b2):
    n, d = x.shape
    e = edge_index.shape[1]
    row = edge_index[0]
    col = edge_index[1]
    zeros = jnp.zeros((n, d), jnp.float32)
    scatter = _make_scatter(n, d, e)

    y1 = _linear(x, W1, b1)
    p1 = scatter(y1, row, col, zeros)
    y2 = _mid(p1, y1, W2, b2)
    p2 = scatter(y2, row, col, zeros)
    return _final(p2, y2)


# speedup vs baseline: 13.2967x; 1.0074x over previous
import functools

import jax
import jax.numpy as jnp
from jax import lax
from jax.experimental import pallas as pl
from jax.experimental.pallas import tpu as pltpu
from jax.experimental.pallas import tpu_sc as plsc

_ROW_BLOCK = 2000


def _linear_body(x_ref, w_ref, b_ref, o_ref):
    o_ref[...] = lax.dot_general(
        x_ref[...], w_ref[...], (((1,), (1,)), ((), ())),
        preferred_element_type=jnp.float32,
        precision=lax.Precision.HIGHEST,
    ) + b_ref[...]


def _linear(x, w, b):
    n, d = x.shape
    br = _ROW_BLOCK
    return pl.pallas_call(
        _linear_body,
        grid=(n // br,),
        in_specs=[
            pl.BlockSpec((br, d), lambda i: (i, 0)),
            pl.BlockSpec((d, d), lambda i: (0, 0)),
            pl.BlockSpec((1, d), lambda i: (0, 0)),
        ],
        out_specs=pl.BlockSpec((br, d), lambda i: (i, 0)),
        out_shape=jax.ShapeDtypeStruct((n, d), jnp.float32),
    )(x, w, b.reshape(1, d))


def _mid_body(p_ref, y_ref, w_ref, b_ref, o_ref):
    h = p_ref[0] + p_ref[1] + y_ref[...]
    h = jnp.maximum(h, 0.0)
    o_ref[...] = lax.dot_general(
        h, w_ref[...], (((1,), (1,)), ((), ())),
        preferred_element_type=jnp.float32,
        precision=lax.Precision.HIGHEST,
    ) + b_ref[...]


def _mid(p, y, w, b):
    n, d = y.shape
    br = _ROW_BLOCK
    return pl.pallas_call(
        _mid_body,
        grid=(n // br,),
        in_specs=[
            pl.BlockSpec((2, br, d), lambda i: (0, i, 0)),
            pl.BlockSpec((br, d), lambda i: (i, 0)),
            pl.BlockSpec((d, d), lambda i: (0, 0)),
            pl.BlockSpec((1, d), lambda i: (0, 0)),
        ],
        out_specs=pl.BlockSpec((br, d), lambda i: (i, 0)),
        out_shape=jax.ShapeDtypeStruct((n, d), jnp.float32),
    )(p, y, w, b.reshape(1, d))


def _final_body(p_ref, y_ref, o_ref):
    z = p_ref[0] + p_ref[1] + y_ref[...]
    m = jnp.max(z, axis=1, keepdims=True)
    s = z - m
    lse = jnp.log(jnp.sum(jnp.exp(s), axis=1, keepdims=True))
    o_ref[...] = s - lse


def _final(p, y):
    n, d = y.shape
    br = _ROW_BLOCK
    return pl.pallas_call(
        _final_body,
        grid=(n // br,),
        in_specs=[
            pl.BlockSpec((2, br, d), lambda i: (0, i, 0)),
            pl.BlockSpec((br, d), lambda i: (i, 0)),
        ],
        out_specs=pl.BlockSpec((br, d), lambda i: (i, 0)),
        out_shape=jax.ShapeDtypeStruct((n, d), jnp.float32),
    )(p, y)


@functools.lru_cache(maxsize=None)
def _make_scatter(n, d, e):
    info = plsc.get_sparse_core_info()
    nc, ns = info.num_cores, info.num_subcores
    k = 80
    epw = e // (nc * ns)
    chunks = epw // k
    nbuf = 2
    slab = (n // (8 * ns)) * 8
    tail = n - slab * ns
    mesh = plsc.VectorSubcoreMesh(core_axis_name="c", subcore_axis_name="s")

    rem = chunks % nbuf
    assert k % 8 == 0 and epw == chunks * k

    @functools.partial(
        pl.kernel, mesh=mesh,
        out_type=jax.ShapeDtypeStruct((nc, n, d), jnp.float32),
        scratch_types=[
            pltpu.VMEM((epw,), jnp.int32),
            pltpu.VMEM((chunks, k), jnp.int32),
            pltpu.VMEM((nbuf, k, d), jnp.float32),
            pltpu.VMEM_SHARED((n, d), jnp.float32),
            pltpu.SemaphoreType.DMA((nbuf,)),
            pltpu.SemaphoreType.DMA((nbuf,)),
        ],
    )
    def scatter(y_hbm, row_hbm, col3_hbm, zeros_hbm, out_hbm,
                ridx_all, cidx_all, rows, acc, gsem, ssem):
        cid = lax.axis_index("c")
        sid = lax.axis_index("s")
        wid = cid * ns + sid
        pltpu.sync_copy(row_hbm.at[pl.ds(wid * epw, epw)], ridx_all)
        pltpu.sync_copy(col3_hbm.at[wid], cidx_all)
        r0 = pl.multiple_of(sid * slab, 8)
        pltpu.sync_copy(zeros_hbm.at[pl.ds(r0, slab)], acc.at[pl.ds(r0, slab)])
        if tail:
            @pl.when(sid == 0)
            def _zero_tail():
                pltpu.sync_copy(zeros_hbm.at[pl.ds(slab * ns, tail)],
                                acc.at[pl.ds(slab * ns, tail)])
        plsc.subcore_barrier()

        def gather(c, j):
            pltpu.async_copy(y_hbm.at[ridx_all.at[pl.ds(c * k, k)]],
                             rows.at[j], gsem.at[j])

        def drain(sem, j):
            pltpu.make_async_copy(y_hbm.at[pl.ds(0, k)], rows.at[j],
                                  sem.at[j]).wait()

        for j in range(nbuf):
            gather(j, j)

        def body(g, carry):
            c = g * nbuf
            for j in range(nbuf):
                drain(gsem, j)
                pltpu.async_copy(rows.at[j], acc.at[cidx_all.at[c + j]],
                                 ssem.at[j], add=True)
            for j in range(nbuf):
                drain(ssem, j)
                gather((c + j + nbuf) % chunks, j)
            return carry

        lax.fori_loop(0, chunks // nbuf, body, 0)
        for j in range(nbuf):
            drain(gsem, j)
            if j < rem:
                pltpu.sync_copy(rows.at[j],
                                acc.at[cidx_all.at[chunks - rem + j]], add=True)

        plsc.subcore_barrier()
        pltpu.sync_copy(acc.at[pl.ds(r0, slab)], out_hbm.at[cid, pl.ds(r0, slab)])
        if tail:
            @pl.when(sid == 0)
            def _write_tail():
                pltpu.sync_copy(acc.at[pl.ds(slab * ns, tail)],
                                out_hbm.at[cid, pl.ds(slab * ns, tail)])

    def run(y, row, col, zeros):
        return scatter(y, row, col.reshape(nc * ns, chunks, k), zeros)

    return run


def kernel(x, edge_index, W1, b1, W2, b2):
    n, d = x.shape
    e = edge_index.shape[1]
    row = edge_index[0]
    col = edge_index[1]
    zeros = jnp.zeros((n, d), jnp.float32)
    scatter = _make_scatter(n, d, e)

    y1 = _linear(x, W1, b1)
    p1 = scatter(y1, row, col, zeros)
    y2 = _mid(p1, y1, W2, b2)
    p2 = scatter(y2, row, col, zeros)
    return _final(p2, y2)


# speedup vs baseline: 14.9487x; 1.1242x over previous
import functools

import jax
import jax.numpy as jnp
from jax import lax
from jax.experimental import pallas as pl
from jax.experimental.pallas import tpu as pltpu
from jax.experimental.pallas import tpu_sc as plsc

_ROW_BLOCK = 2000


def _linear_body(x_ref, w_ref, b_ref, o_ref):
    o_ref[...] = lax.dot_general(
        x_ref[...], w_ref[...], (((1,), (1,)), ((), ())),
        preferred_element_type=jnp.float32,
        precision=lax.Precision.HIGHEST,
    ) + b_ref[...]


def _linear(x, w, b):
    n, d = x.shape
    br = _ROW_BLOCK
    return pl.pallas_call(
        _linear_body,
        grid=(n // br,),
        in_specs=[
            pl.BlockSpec((br, d), lambda i: (i, 0)),
            pl.BlockSpec((d, d), lambda i: (0, 0)),
            pl.BlockSpec((1, d), lambda i: (0, 0)),
        ],
        out_specs=pl.BlockSpec((br, d), lambda i: (i, 0)),
        out_shape=jax.ShapeDtypeStruct((n, d), jnp.float32),
    )(x, w, b.reshape(1, d))


def _mid_body(p_ref, y_ref, w_ref, b_ref, o_ref):
    h = p_ref[0] + p_ref[1] + y_ref[...]
    h = jnp.maximum(h, 0.0)
    o_ref[...] = lax.dot_general(
        h, w_ref[...], (((1,), (1,)), ((), ())),
        preferred_element_type=jnp.float32,
        precision=lax.Precision.HIGHEST,
    ) + b_ref[...]


def _mid(p, y, w, b):
    n, d = y.shape
    br = _ROW_BLOCK
    return pl.pallas_call(
        _mid_body,
        grid=(n // br,),
        in_specs=[
            pl.BlockSpec((2, br, d), lambda i: (0, i, 0)),
            pl.BlockSpec((br, d), lambda i: (i, 0)),
            pl.BlockSpec((d, d), lambda i: (0, 0)),
            pl.BlockSpec((1, d), lambda i: (0, 0)),
        ],
        out_specs=pl.BlockSpec((br, d), lambda i: (i, 0)),
        out_shape=jax.ShapeDtypeStruct((n, d), jnp.float32),
    )(p, y, w, b.reshape(1, d))


def _final_body(p_ref, y_ref, o_ref):
    z = p_ref[0] + p_ref[1] + y_ref[...]
    m = jnp.max(z, axis=1, keepdims=True)
    s = z - m
    lse = jnp.log(jnp.sum(jnp.exp(s), axis=1, keepdims=True))
    o_ref[...] = s - lse


def _final(p, y):
    n, d = y.shape
    br = _ROW_BLOCK
    return pl.pallas_call(
        _final_body,
        grid=(n // br,),
        in_specs=[
            pl.BlockSpec((2, br, d), lambda i: (0, i, 0)),
            pl.BlockSpec((br, d), lambda i: (i, 0)),
        ],
        out_specs=pl.BlockSpec((br, d), lambda i: (i, 0)),
        out_shape=jax.ShapeDtypeStruct((n, d), jnp.float32),
    )(p, y)


@functools.lru_cache(maxsize=None)
def _make_scatter(n, d, e):
    info = plsc.get_sparse_core_info()
    nc, ns = info.num_cores, info.num_subcores
    k = 80
    epw = e // (nc * ns)
    chunks = epw // k
    nbuf = 4
    slab = (n // (8 * ns)) * 8
    tail = n - slab * ns
    mesh = plsc.VectorSubcoreMesh(core_axis_name="c", subcore_axis_name="s")

    rem = chunks % nbuf
    assert k % 8 == 0 and epw == chunks * k

    @functools.partial(
        pl.kernel, mesh=mesh,
        out_type=jax.ShapeDtypeStruct((nc, n, d), jnp.float32),
        scratch_types=[
            [pltpu.VMEM((k,), jnp.int32) for _ in range(nbuf)],
            [pltpu.VMEM((k,), jnp.int32) for _ in range(nbuf)],
            pltpu.VMEM((nbuf, k, d), jnp.float32),
            pltpu.VMEM_SHARED((n, d), jnp.float32),
            pltpu.SemaphoreType.DMA((nbuf,)),
            pltpu.SemaphoreType.DMA((nbuf,)),
            pltpu.SemaphoreType.DMA((nbuf,)),
            pltpu.SemaphoreType.DMA((nbuf,)),
        ],
    )
    def scatter(y_hbm, row_hbm, col_hbm, zeros_hbm, out_hbm,
                rbufs, cbufs, rows, acc, rsem, csem, gsem, ssem):
        cid = lax.axis_index("c")
        sid = lax.axis_index("s")
        wid = cid * ns + sid
        base = wid * epw
        r0 = pl.multiple_of(sid * slab, 8)
        pltpu.sync_copy(zeros_hbm.at[pl.ds(r0, slab)], acc.at[pl.ds(r0, slab)])
        if tail:
            @pl.when(sid == 0)
            def _zero_tail():
                pltpu.sync_copy(zeros_hbm.at[pl.ds(slab * ns, tail)],
                                acc.at[pl.ds(slab * ns, tail)])
        plsc.subcore_barrier()

        def idx_dma(c, j):
            off = pl.multiple_of(base + c * k, 8)
            pltpu.async_copy(row_hbm.at[pl.ds(off, k)], rbufs[j], rsem.at[j])
            pltpu.async_copy(col_hbm.at[pl.ds(off, k)], cbufs[j], csem.at[j])

        def drain_idx(sem, buf, j):
            pltpu.make_async_copy(row_hbm.at[pl.ds(0, k)], buf, sem.at[j]).wait()

        def drain_rows(sem, j):
            pltpu.make_async_copy(y_hbm.at[pl.ds(0, k)], rows.at[j],
                                  sem.at[j]).wait()

        def gather(j):
            pltpu.async_copy(y_hbm.at[rbufs[j]], rows.at[j], gsem.at[j])

        for j in range(nbuf):
            idx_dma(j, j)

        def body(g, carry):
            c = g * nbuf
            for j in range(nbuf):
                drain_idx(rsem, rbufs[j], j)
                gather(j)
            for j in range(nbuf):
                drain_rows(gsem, j)
                drain_idx(csem, cbufs[j], j)
                pltpu.async_copy(rows.at[j], acc.at[cbufs[j]],
                                 ssem.at[j], add=True)
            for j in range(nbuf):
                drain_rows(ssem, j)
                idx_dma((c + j + nbuf) % chunks, j)
            return carry

        lax.fori_loop(0, chunks // nbuf, body, 0)
        for j in range(nbuf):
            drain_idx(rsem, rbufs[j], j)
            drain_idx(csem, cbufs[j], j)
            if j < rem:
                gather(j)
        for j in range(rem):
            drain_rows(gsem, j)
            pltpu.sync_copy(rows.at[j], acc.at[cbufs[j]], add=True)

        plsc.subcore_barrier()
        pltpu.sync_copy(acc.at[pl.ds(r0, slab)], out_hbm.at[cid, pl.ds(r0, slab)])
        if tail:
            @pl.when(sid == 0)
            def _write_tail():
                pltpu.sync_copy(acc.at[pl.ds(slab * ns, tail)],
                                out_hbm.at[cid, pl.ds(slab * ns, tail)])

    def run(y, row, col, zeros):
        return scatter(y, row, col, zeros)

    return run


def kernel(x, edge_index, W1, b1, W2, b2):
    n, d = x.shape
    e = edge_index.shape[1]
    row = edge_index[0]
    col = edge_index[1]
    zeros = jnp.zeros((n, d), jnp.float32)
    scatter = _make_scatter(n, d, e)

    y1 = _linear(x, W1, b1)
    p1 = scatter(y1, row, col, zeros)
    y2 = _mid(p1, y1, W2, b2)
    p2 = scatter(y2, row, col, zeros)
    return _final(p2, y2)
